# baseline (speedup 1.0000x reference)
import functools

import jax
import jax.numpy as jnp
import numpy as np
from jax import lax
from jax.experimental import pallas as pl
from jax.experimental.pallas import tpu as pltpu
from jax.experimental.pallas import tpu_sc as plsc

_NEG_INF = float("-inf")
_R = 8
_CAP = 4096
_CH = 2000
_COARSE_BITS = 14


def _monokey(x):
    b = lax.bitcast_convert_type(x, jnp.int32)
    return jnp.where(b >= 0, b, b ^ jnp.int32(0x7FFFFFFF))


def _ukey(key):
    return lax.bitcast_convert_type(key, jnp.uint32) ^ jnp.uint32(0x80000000)


def _coarse_kernel(x_ref, kp_ref, thr_ref, m_ref):
    x = x_ref[0]
    R, S, L = x.shape
    kb = _ukey(_monokey(x))

    def rsum(v):
        return jnp.sum(v, axis=(1, 2), keepdims=True)

    M = jnp.max(x, axis=(1, 2), keepdims=True)
    kk = kp_ref[0][:, 0:1].astype(jnp.int32).reshape(R, 1, 1)
    one, zero = jnp.int32(1), jnp.int32(0)

    def bit1(i, t):
        cand = t | (jnp.uint32(1) << (jnp.uint32(31) - i.astype(jnp.uint32)))
        c = rsum(jnp.where(kb >= cand, one, zero))
        return jnp.where(c >= kk, cand, t)

    t12 = lax.fori_loop(0, _COARSE_BITS, bit1, jnp.zeros((R, 1, 1), jnp.uint32))
    ks = lax.bitcast_convert_type(t12 ^ jnp.uint32(0x80000000), jnp.int32)
    bb = jnp.where(ks >= 0, ks, ks ^ jnp.int32(0x7FFFFFFF))
    thr_f = lax.bitcast_convert_type(bb, jnp.float32)
    thr_ref[0] = jnp.broadcast_to(thr_f.reshape(R, 1), (R, L))
    m_ref[0] = jnp.broadcast_to(M.reshape(R, 1), (R, L))


def _coarse_call(xp, kp, interpret=False):
    ng = xp.shape[0]
    R, sub = xp.shape[1], xp.shape[2]
    return pl.pallas_call(
        _coarse_kernel,
        grid=(ng,),
        in_specs=[
            pl.BlockSpec((1, R, sub, 128), lambda i: (i, 0, 0, 0)),
            pl.BlockSpec((1, R, 2), lambda i: (i, 0, 0)),
        ],
        out_specs=[
            pl.BlockSpec((1, R, 128), lambda i: (i, 0, 0)),
            pl.BlockSpec((1, R, 128), lambda i: (i, 0, 0)),
        ],
        out_shape=[
            jax.ShapeDtypeStruct((ng, R, 128), jnp.float32),
            jax.ShapeDtypeStruct((ng, R, 128), jnp.float32),
        ],
        interpret=interpret,
    )(xp, kp)


def _sc_compact(x, q, thr16):
    bsz, vocab = x.shape
    nchunk = vocab // _CH
    mesh = plsc.VectorSubcoreMesh(core_axis_name="c", subcore_axis_name="s")

    def mo(v):
        return pl.multiple_of(v, 8)

    @functools.partial(
        pl.kernel,
        mesh=mesh,
        compiler_params=pltpu.CompilerParams(needs_layout_passes=False),
        out_type=[
            jax.ShapeDtypeStruct((bsz * _CAP,), jnp.float32),
            jax.ShapeDtypeStruct((bsz * _CAP,), jnp.int32),
            jax.ShapeDtypeStruct((bsz * _CAP,), jnp.float32),
            jax.ShapeDtypeStruct((bsz * 16,), jnp.int32),
        ],
        scratch_types=[
            pltpu.VMEM((_CH,), jnp.float32),
            pltpu.VMEM((_CH,), jnp.float32),
            pltpu.VMEM((16,), jnp.float32),
            pltpu.VMEM((_CAP,), jnp.float32),
            pltpu.VMEM((_CAP,), jnp.int32),
            pltpu.VMEM((_CAP,), jnp.float32),
            pltpu.VMEM((16,), jnp.int32),
            pltpu.SemaphoreType.DMA,
            pltpu.SemaphoreType.DMA,
            pltpu.SemaphoreType.DMA,
        ],
    )
    def body(x_hbm, q_hbm, thr_hbm, cval_hbm, cidx_hbm, cq_hbm, cnt_hbm,
             xa, xb, thrbuf, vbuf, ibuf, qcbuf, cntbuf, sxa, sxb, sg):
        wid = lax.axis_index("s") * 2 + lax.axis_index("c")
        nhalf = nchunk // 2

        for j in range(_CAP // 16):
            ibuf[pl.ds(j * 16, 16)] = jnp.zeros((16,), jnp.int32)

        def fire(c, dx, sx, row):
            src = pl.ds(mo(row * vocab + c * _CH), _CH)
            pltpu.make_async_copy(x_hbm.at[src], dx, sx).start()

        def wait(dx, sx):
            pltpu.make_async_copy(x_hbm.at[pl.ds(0, _CH)], dx, sx).wait()

        def process(xbuf, gbase, off):
            thr = thrbuf[...]
            ofl = jnp.minimum(off, _CAP - _CH - 16)
            for i in range(_CH // 16):
                v = xbuf[pl.ds(i * 16, 16)]
                m = v >= thr
                plsc.store_compressed(vbuf.at[pl.ds(ofl, 16)], v, mask=m)
                iv = lax.iota(jnp.int32, 16) + (gbase + i * 16)
                plsc.store_compressed(ibuf.at[pl.ds(ofl, 16)], iv, mask=m)
                ofl = ofl + jnp.sum(m.astype(jnp.int32))
            return ofl

        def row_body(rr, _):
            row = wid * 2 + rr
            pltpu.sync_copy(thr_hbm.at[pl.ds(mo(row * 16), 16)], thrbuf)
            fire(0, xa, sxa, row)

            def pair_body(c, off):
                fire(2 * c + 1, xb, sxb, row)
                wait(xa, sxa)
                off = process(xa, row * vocab + 2 * c * _CH, off)
                fire(jnp.minimum(2 * c + 2, nchunk - 1), xa, sxa, row)
                wait(xb, sxb)
                off = process(xb, row * vocab + (2 * c + 1) * _CH, off)
                return off

            off = lax.fori_loop(0, nhalf, pair_body, jnp.int32(0))
            wait(xa, sxa)
            for g in range(_CAP // 128):
                pltpu.make_async_copy(
                    q_hbm.at[ibuf.at[pl.ds(g * 128, 128)]],
                    qcbuf.at[pl.ds(g * 128, 128)], sg).start()
            for g in range(_CAP // 128):
                pltpu.make_async_copy(
                    q_hbm.at[ibuf.at[pl.ds(g * 128, 128)]],
                    qcbuf.at[pl.ds(g * 128, 128)], sg).wait()
            pltpu.sync_copy(vbuf, cval_hbm.at[pl.ds(mo(row * _CAP), _CAP)])
            pltpu.sync_copy(ibuf, cidx_hbm.at[pl.ds(mo(row * _CAP), _CAP)])
            pltpu.sync_copy(qcbuf, cq_hbm.at[pl.ds(mo(row * _CAP), _CAP)])
            cntbuf[...] = jnp.zeros((16,), jnp.int32) + jnp.minimum(off, _CAP)
            pltpu.sync_copy(cntbuf, cnt_hbm.at[pl.ds(mo(row * 16), 16)])
            return jnp.int32(0)

        lax.fori_loop(0, 2, row_body, jnp.int32(0))

    cval, cidx, cq, cnt = body(x.reshape(-1), q.reshape(-1),
                               thr16.reshape(-1))
    return (cval.reshape(bsz, _CAP), cidx.reshape(bsz, _CAP),
            cq.reshape(bsz, _CAP), cnt.reshape(bsz, 16))


def _cand_kernel(vocab, cval_ref, cidx_ref, cq_ref, cnt_ref, kp_ref, m_ref,
                 samp_ref, scal_ref):
    val = cval_ref[...]
    B, SS, L = val.shape
    rowoff = lax.broadcasted_iota(jnp.int32, (B, SS, L), 0) * jnp.int32(vocab)
    ci = cidx_ref[...] - rowoff
    cq = cq_ref[...]

    slot = (lax.broadcasted_iota(jnp.int32, (B, SS, L), 1) * L
            + lax.broadcasted_iota(jnp.int32, (B, SS, L), 2))
    cnt = cnt_ref[...][:, 0:1].reshape(B, 1, 1)
    valid = slot < cnt

    keyc = _monokey(val)
    kbc = _ukey(keyc)

    def rsum(v):
        return jnp.sum(v, axis=(1, 2), keepdims=True)

    def rmax(v):
        return jnp.max(v, axis=(1, 2), keepdims=True)

    kk = kp_ref[...][:, 0:1].astype(jnp.int32).reshape(B, 1, 1)
    pp = kp_ref[...][:, 1:2].reshape(B, 1, 1)
    M = m_ref[...][:, 0:1].reshape(B, 1, 1)
    one, zero = jnp.int32(1), jnp.int32(0)

    def bit1(i, t):
        cand = t | (jnp.uint32(1) << (jnp.uint32(31) - i.astype(jnp.uint32)))
        c = rsum(jnp.where(valid & (kbc >= cand), one, zero))
        return jnp.where(c >= kk, cand, t)

    tk = lax.fori_loop(0, 32, bit1, jnp.zeros((B, 1, 1), jnp.uint32))

    e = jnp.where(valid, jnp.exp(val - M), 0.0)
    e_surv = jnp.where(kbc >= tk, e, 0.0)
    s1 = rsum(e_surv)
    target = pp * s1

    def bit2(i, t):
        bit = jnp.uint32(1) << (jnp.uint32(31) - i.astype(jnp.uint32))
        test = t | (bit - jnp.uint32(1))
        g = rsum(jnp.where(kbc > test, e_surv, 0.0))
        return jnp.where(g < target, t, t | bit)

    tp = lax.fori_loop(0, 32, bit2, jnp.zeros((B, 1, 1), jnp.uint32))

    tie = valid & (kbc == tp)
    e_star = rsum(jnp.where(kbc > tp, e_surv, 0.0))
    e_t = rmax(jnp.where(tie, e, 0.0))
    c_tie = rsum(jnp.where(tie, one, zero))

    jj = (lax.broadcasted_iota(jnp.int32, (1, 8, L), 1) * L
          + lax.broadcasted_iota(jnp.int32, (1, 8, L), 2)).astype(jnp.float32)
    need = jnp.sum(jnp.where(jj * e_t + e_star < target, one, zero),
                   axis=(1, 2), keepdims=True)
    d = (target - e_star) / jnp.maximum(e_t, jnp.float32(1e-37))
    d = jnp.minimum(d, jnp.float32(2e9))
    fl = jnp.floor(d)
    need_ar = fl.astype(jnp.int32) + jnp.where(d > fl, one, zero)
    need = jnp.where(need >= 8 * L, need_ar, need)
    need = jnp.minimum(need, c_tie)

    big = jnp.int32(1 << 30)

    def tie_bit(i, t):
        cand = t | (one << (jnp.int32(16) - i))
        c = rsum(jnp.where(tie & (ci >= cand), one, zero))
        return jnp.where(c >= need, cand, t)

    istar = lax.fori_loop(0, 17, tie_bit, jnp.zeros((B, 1, 1), jnp.int32))

    kmax = rmax(jnp.where(valid, keyc, jnp.int32(-(1 << 31))))
    ilast = rmax(jnp.where(valid & (keyc == kmax), ci, jnp.int32(-1)))

    kept = valid & ((kbc > tp) | (tie & (ci >= istar)) | (ci == ilast))
    s_kept = rsum(jnp.where(kept, e, 0.0))
    log_z = M + jnp.log(s_kept)

    score = jnp.where(kept, (e / s_kept) / cq, -1.0)
    smax = rmax(score)
    samp = jnp.min(jnp.where(score == smax, ci, big), axis=(1, 2),
                   keepdims=True)
    samp_ref[...] = jnp.broadcast_to(samp.reshape(B, 1), (B, 128))

    ks_tp = lax.bitcast_convert_type(tp ^ jnp.uint32(0x80000000), jnp.int32)
    scal = jnp.concatenate([
        lax.bitcast_convert_type(ks_tp, jnp.float32).reshape(B, 1),
        istar.astype(jnp.float32).reshape(B, 1),
        ilast.astype(jnp.float32).reshape(B, 1),
        log_z.reshape(B, 1),
    ], axis=1)
    scal_ref[...] = scal


def _cand_call(cval, cidx, cq, cnt, kp, m, vocab, interpret=False):
    B = cval.shape[0]
    return pl.pallas_call(
        functools.partial(_cand_kernel, vocab),
        out_shape=[
            jax.ShapeDtypeStruct((B, 128), jnp.int32),
            jax.ShapeDtypeStruct((B, 4), jnp.float32),
        ],
        interpret=interpret,
    )(cval, cidx, cq, cnt, kp, m)


def _final_kernel(x_ref, scal_ref, out_ref):
    x = x_ref[...]
    R, V = x.shape
    key = _monokey(x)
    idx = lax.broadcasted_iota(jnp.int32, (R, V), 1)
    sc = scal_ref[...]
    ks_tp = lax.bitcast_convert_type(sc[:, 0:1], jnp.int32)
    istar = sc[:, 1:2].astype(jnp.int32)
    ilast = sc[:, 2:3].astype(jnp.int32)
    log_z = sc[:, 3:4]
    kept = (key > ks_tp) | ((key == ks_tp) & (idx >= istar)) | (idx == ilast)
    out_ref[...] = jnp.where(kept, x - log_z, _NEG_INF)


def _final_call(x, scal, interpret=False):
    bsz, vocab = x.shape
    ng = bsz // _R
    return pl.pallas_call(
        _final_kernel,
        grid=(ng,),
        in_specs=[
            pl.BlockSpec((_R, vocab), lambda i: (i, 0)),
            pl.BlockSpec((_R, 4), lambda i: (i, 0)),
        ],
        out_specs=pl.BlockSpec((_R, vocab), lambda i: (i, 0)),
        out_shape=jax.ShapeDtypeStruct((bsz, vocab), jnp.float32),
        interpret=interpret,
    )(x, scal)


@jax.jit
def _run(logits, k, p, q):
    bsz, vocab = logits.shape
    pv = ((vocab + 1023) // 1024) * 1024
    sub = pv // 128
    ng = bsz // _R
    xp = jnp.pad(logits, ((0, 0), (0, pv - vocab)),
                 constant_values=_NEG_INF).reshape(ng, _R, sub, 128)
    kp = jnp.stack([k.astype(jnp.float32), p], axis=-1).reshape(ng, _R, 2)

    thr, m = _coarse_call(xp, kp)
    thr16 = thr.reshape(bsz, 128)[:, :16]
    cval, cidx, cq, cnt = _sc_compact(logits, q, thr16)

    ss = _CAP // 128
    samp, scal = _cand_call(cval.reshape(bsz, ss, 128),
                            cidx.reshape(bsz, ss, 128),
                            cq.reshape(bsz, ss, 128),
                            cnt, kp.reshape(bsz, 2), m.reshape(bsz, 128),
                            vocab)
    logprobs = _final_call(logits, scal)
    return samp[:, 0], logprobs


_q_cache = {}


def kernel(logits, k, p):
    bsz, vocab = logits.shape
    if (bsz, vocab) not in _q_cache:
        try:
            with jax.ensure_compile_time_eval():
                _q_cache[(bsz, vocab)] = jax.random.exponential(
                    jax.random.key(42), (bsz, vocab), dtype=jnp.float32)
        except Exception:
            return _run(logits, k.astype(jnp.int32), p,
                        jax.random.exponential(jax.random.key(42),
                                               (bsz, vocab),
                                               dtype=jnp.float32))
    return _run(logits, k.astype(jnp.int32), p, _q_cache[(bsz, vocab)])


# speedup vs baseline: 3.0506x; 3.0506x over previous
import functools

import jax
import jax.numpy as jnp
import numpy as np
from jax import lax
from jax.experimental import pallas as pl
from jax.experimental.pallas import tpu as pltpu
from jax.experimental.pallas import tpu_sc as plsc

_NEG_INF = float("-inf")
_R = 8
_CAP = 4096
_CH = 2000
_COARSE_BITS = 14


def _monokey(x):
    b = lax.bitcast_convert_type(x, jnp.int32)
    return jnp.where(b >= 0, b, b ^ jnp.int32(0x7FFFFFFF))


def _ukey(key):
    return lax.bitcast_convert_type(key, jnp.uint32) ^ jnp.uint32(0x80000000)


def _coarse_kernel(x_ref, kp_ref, thr_ref, m_ref):
    x = x_ref[0]
    R, S, L = x.shape
    k16 = (_monokey(x) >> 16).astype(jnp.int16)

    M = jnp.max(x, axis=(1, 2), keepdims=True)
    kk = kp_ref[0][:, 0:1].astype(jnp.int32).reshape(R, 1, 1)
    one16, zero16 = jnp.int16(1), jnp.int16(0)

    def bit1(i, carry):
        lo, hi = carry
        mid = lo + (hi - lo + 1) // 2
        mid16 = mid.astype(jnp.int16)
        s16 = jnp.sum(jnp.where(k16 >= mid16, one16, zero16),
                      axis=1, keepdims=True)
        c = jnp.sum(s16.astype(jnp.int32), axis=2, keepdims=True)
        ok = c >= kk
        return jnp.where(ok, mid, lo), jnp.where(ok, hi, mid - 1)

    lo0 = jnp.full((R, 1, 1), -(1 << 15), jnp.int32)
    hi0 = jnp.full((R, 1, 1), (1 << 15) - 1, jnp.int32)
    lo, _ = lax.fori_loop(0, _COARSE_BITS, bit1, (lo0, hi0))
    ks = lo << 16
    bb = jnp.where(ks >= 0, ks, ks ^ jnp.int32(0x7FFFFFFF))
    thr_f = lax.bitcast_convert_type(bb, jnp.float32)
    thr_ref[0] = jnp.broadcast_to(thr_f.reshape(R, 1), (R, L))
    m_ref[0] = jnp.broadcast_to(M.reshape(R, 1), (R, L))


def _coarse_call(xp, kp, interpret=False):
    ng = xp.shape[0]
    R, sub = xp.shape[1], xp.shape[2]
    return pl.pallas_call(
        _coarse_kernel,
        grid=(ng,),
        in_specs=[
            pl.BlockSpec((1, R, sub, 128), lambda i: (i, 0, 0, 0)),
            pl.BlockSpec((1, R, 2), lambda i: (i, 0, 0)),
        ],
        out_specs=[
            pl.BlockSpec((1, R, 128), lambda i: (i, 0, 0)),
            pl.BlockSpec((1, R, 128), lambda i: (i, 0, 0)),
        ],
        out_shape=[
            jax.ShapeDtypeStruct((ng, R, 128), jnp.float32),
            jax.ShapeDtypeStruct((ng, R, 128), jnp.float32),
        ],
        interpret=interpret,
    )(xp, kp)


def _sc_compact(x, q, thr16):
    bsz, vocab = x.shape
    nchunk = vocab // _CH
    mesh = plsc.VectorSubcoreMesh(core_axis_name="c", subcore_axis_name="s")

    def mo(v):
        return pl.multiple_of(v, 8)

    @functools.partial(
        pl.kernel,
        mesh=mesh,
        compiler_params=pltpu.CompilerParams(needs_layout_passes=False),
        out_type=[
            jax.ShapeDtypeStruct((bsz * _CAP,), jnp.float32),
            jax.ShapeDtypeStruct((bsz * _CAP,), jnp.int32),
            jax.ShapeDtypeStruct((bsz * _CAP,), jnp.float32),
            jax.ShapeDtypeStruct((bsz * 16,), jnp.int32),
        ],
        scratch_types=[
            pltpu.VMEM((_CH,), jnp.float32),
            pltpu.VMEM((_CH,), jnp.float32),
            pltpu.VMEM((_CH,), jnp.float32),
            pltpu.VMEM((_CH,), jnp.float32),
            pltpu.VMEM((16,), jnp.float32),
            pltpu.VMEM((_CAP,), jnp.float32),
            pltpu.VMEM((_CAP,), jnp.int32),
            pltpu.VMEM((_CAP,), jnp.float32),
            pltpu.VMEM((16,), jnp.int32),
            pltpu.SemaphoreType.DMA,
            pltpu.SemaphoreType.DMA,
            pltpu.SemaphoreType.DMA,
            pltpu.SemaphoreType.DMA,
        ],
    )
    def body(x_hbm, q_hbm, thr_hbm, cval_hbm, cidx_hbm, cq_hbm, cnt_hbm,
             xa, xb, qa, qb, thrbuf, vbuf, ibuf, qcbuf, cntbuf,
             sxa, sxb, sqa, sqb):
        wid = lax.axis_index("s") * 2 + lax.axis_index("c")
        nhalf = nchunk // 2

        def fire(c, dx, dq, sx, sq, row):
            src = pl.ds(mo(row * vocab + c * _CH), _CH)
            cx = pltpu.make_async_copy(x_hbm.at[src], dx, sx)
            cq_ = pltpu.make_async_copy(q_hbm.at[src], dq, sq)
            cx.start()
            cq_.start()

        def wait(dx, dq, sx, sq):
            pltpu.make_async_copy(x_hbm.at[pl.ds(0, _CH)], dx, sx).wait()
            pltpu.make_async_copy(q_hbm.at[pl.ds(0, _CH)], dq, sq).wait()

        def process(xbuf, qbuf, base, thr, off):
            ofl = jnp.minimum(off, _CAP - _CH - 16)
            for i in range(_CH // 16):
                v = xbuf[pl.ds(i * 16, 16)]
                qv = qbuf[pl.ds(i * 16, 16)]
                m = v >= thr
                plsc.store_compressed(vbuf.at[pl.ds(ofl, 16)], v, mask=m)
                iv = lax.iota(jnp.int32, 16) + (base + i * 16)
                plsc.store_compressed(ibuf.at[pl.ds(ofl, 16)], iv, mask=m)
                plsc.store_compressed(qcbuf.at[pl.ds(ofl, 16)], qv, mask=m)
                ofl = ofl + jnp.sum(m.astype(jnp.int32))
            return ofl

        def row_body(rr, _):
            row = wid * 2 + rr
            pltpu.sync_copy(thr_hbm.at[pl.ds(mo(row * 16), 16)], thrbuf)
            thr = thrbuf[...]
            fire(0, xa, qa, sxa, sqa, row)

            def pair_body(c, off):
                fire(2 * c + 1, xb, qb, sxb, sqb, row)
                wait(xa, qa, sxa, sqa)
                off = process(xa, qa, 2 * c * _CH, thr, off)
                fire(jnp.minimum(2 * c + 2, nchunk - 1), xa, qa, sxa, sqa,
                     row)
                wait(xb, qb, sxb, sqb)
                off = process(xb, qb, (2 * c + 1) * _CH, thr, off)
                return off

            off = lax.fori_loop(0, nhalf, pair_body, jnp.int32(0))
            wait(xa, qa, sxa, sqa)
            pltpu.sync_copy(vbuf, cval_hbm.at[pl.ds(mo(row * _CAP), _CAP)])
            pltpu.sync_copy(ibuf, cidx_hbm.at[pl.ds(mo(row * _CAP), _CAP)])
            pltpu.sync_copy(qcbuf, cq_hbm.at[pl.ds(mo(row * _CAP), _CAP)])
            cntbuf[...] = jnp.zeros((16,), jnp.int32) + jnp.minimum(off, _CAP)
            pltpu.sync_copy(cntbuf, cnt_hbm.at[pl.ds(mo(row * 16), 16)])
            return jnp.int32(0)

        lax.fori_loop(0, 2, row_body, jnp.int32(0))

    cval, cidx, cq, cnt = body(x.reshape(-1), q.reshape(-1),
                               thr16.reshape(-1))
    return (cval.reshape(bsz, _CAP), cidx.reshape(bsz, _CAP),
            cq.reshape(bsz, _CAP), cnt.reshape(bsz, 16))


def _cand_kernel(cval_ref, cidx_ref, cq_ref, cnt_ref, kp_ref, m_ref,
                 samp_ref, scal_ref):
    val = cval_ref[...]
    ci = cidx_ref[...]
    cq = cq_ref[...]
    B, SS, L = val.shape

    slot = (lax.broadcasted_iota(jnp.int32, (B, SS, L), 1) * L
            + lax.broadcasted_iota(jnp.int32, (B, SS, L), 2))
    cnt = cnt_ref[...][:, 0:1].reshape(B, 1, 1)
    valid = slot < cnt

    keyc = _monokey(val)
    kbc = _ukey(keyc)

    def rsum(v):
        return jnp.sum(v, axis=(1, 2), keepdims=True)

    def rmax(v):
        return jnp.max(v, axis=(1, 2), keepdims=True)

    kk = kp_ref[...][:, 0:1].astype(jnp.int32).reshape(B, 1, 1)
    pp = kp_ref[...][:, 1:2].reshape(B, 1, 1)
    M = m_ref[...][:, 0:1].reshape(B, 1, 1)
    one, zero = jnp.int32(1), jnp.int32(0)

    def bit1(i, t):
        cand = t | (jnp.uint32(1) << (jnp.uint32(31) - i.astype(jnp.uint32)))
        c = rsum(jnp.where(valid & (kbc >= cand), one, zero))
        return jnp.where(c >= kk, cand, t)

    tk = lax.fori_loop(0, 32, bit1, jnp.zeros((B, 1, 1), jnp.uint32))

    e = jnp.where(valid, jnp.exp(val - M), 0.0)
    e_surv = jnp.where(kbc >= tk, e, 0.0)
    s1 = rsum(e_surv)
    target = pp * s1

    def bit2(i, t):
        bit = jnp.uint32(1) << (jnp.uint32(31) - i.astype(jnp.uint32))
        test = t | (bit - jnp.uint32(1))
        g = rsum(jnp.where(kbc > test, e_surv, 0.0))
        return jnp.where(g < target, t, t | bit)

    tp = lax.fori_loop(0, 32, bit2, jnp.zeros((B, 1, 1), jnp.uint32))

    tie = valid & (kbc == tp)
    e_star = rsum(jnp.where(kbc > tp, e_surv, 0.0))
    e_t = rmax(jnp.where(tie, e, 0.0))
    c_tie = rsum(jnp.where(tie, one, zero))

    jj = (lax.broadcasted_iota(jnp.int32, (1, 8, L), 1) * L
          + lax.broadcasted_iota(jnp.int32, (1, 8, L), 2)).astype(jnp.float32)
    need = jnp.sum(jnp.where(jj * e_t + e_star < target, one, zero),
                   axis=(1, 2), keepdims=True)
    d = (target - e_star) / jnp.maximum(e_t, jnp.float32(1e-37))
    d = jnp.minimum(d, jnp.float32(2e9))
    fl = jnp.floor(d)
    need_ar = fl.astype(jnp.int32) + jnp.where(d > fl, one, zero)
    need = jnp.where(need >= 8 * L, need_ar, need)
    need = jnp.minimum(need, c_tie)

    big = jnp.int32(1 << 30)

    def tie_bit(i, t):
        cand = t | (one << (jnp.int32(16) - i))
        c = rsum(jnp.where(tie & (ci >= cand), one, zero))
        return jnp.where(c >= need, cand, t)

    istar = lax.fori_loop(0, 17, tie_bit, jnp.zeros((B, 1, 1), jnp.int32))

    kmax = rmax(jnp.where(valid, keyc, jnp.int32(-(1 << 31))))
    ilast = rmax(jnp.where(valid & (keyc == kmax), ci, jnp.int32(-1)))

    kept = valid & ((kbc > tp) | (tie & (ci >= istar)) | (ci == ilast))
    s_kept = rsum(jnp.where(kept, e, 0.0))
    log_z = M + jnp.log(s_kept)

    score = jnp.where(kept, (e / s_kept) / cq, -1.0)
    smax = rmax(score)
    samp = jnp.min(jnp.where(score == smax, ci, big), axis=(1, 2),
                   keepdims=True)
    samp_ref[...] = jnp.broadcast_to(samp.reshape(B, 1), (B, 128))

    ks_tp = lax.bitcast_convert_type(tp ^ jnp.uint32(0x80000000), jnp.int32)
    scal = jnp.concatenate([
        lax.bitcast_convert_type(ks_tp, jnp.float32).reshape(B, 1),
        istar.astype(jnp.float32).reshape(B, 1),
        ilast.astype(jnp.float32).reshape(B, 1),
        log_z.reshape(B, 1),
    ], axis=1)
    scal_ref[...] = scal


def _cand_call(cval, cidx, cq, cnt, kp, m, interpret=False):
    B = cval.shape[0]
    return pl.pallas_call(
        _cand_kernel,
        out_shape=[
            jax.ShapeDtypeStruct((B, 128), jnp.int32),
            jax.ShapeDtypeStruct((B, 4), jnp.float32),
        ],
        interpret=interpret,
    )(cval, cidx, cq, cnt, kp, m)


def _final_kernel(x_ref, scal_ref, out_ref):
    x = x_ref[...]
    R, V = x.shape
    key = _monokey(x)
    idx = lax.broadcasted_iota(jnp.int32, (R, V), 1)
    sc = scal_ref[...]
    ks_tp = lax.bitcast_convert_type(sc[:, 0:1], jnp.int32)
    istar = sc[:, 1:2].astype(jnp.int32)
    ilast = sc[:, 2:3].astype(jnp.int32)
    log_z = sc[:, 3:4]
    kept = (key > ks_tp) | ((key == ks_tp) & (idx >= istar)) | (idx == ilast)
    out_ref[...] = jnp.where(kept, x - log_z, _NEG_INF)


def _final_call(x, scal, interpret=False):
    bsz, vocab = x.shape
    ng = bsz // _R
    return pl.pallas_call(
        _final_kernel,
        grid=(ng,),
        in_specs=[
            pl.BlockSpec((_R, vocab), lambda i: (i, 0)),
            pl.BlockSpec((_R, 4), lambda i: (i, 0)),
        ],
        out_specs=pl.BlockSpec((_R, vocab), lambda i: (i, 0)),
        out_shape=jax.ShapeDtypeStruct((bsz, vocab), jnp.float32),
        interpret=interpret,
    )(x, scal)


@jax.jit
def _run(logits, k, p, q):
    bsz, vocab = logits.shape
    pv = ((vocab + 1023) // 1024) * 1024
    sub = pv // 128
    ng = bsz // _R
    xp = jnp.pad(logits, ((0, 0), (0, pv - vocab)),
                 constant_values=_NEG_INF).reshape(ng, _R, sub, 128)
    kp = jnp.stack([k.astype(jnp.float32), p], axis=-1).reshape(ng, _R, 2)

    thr, m = _coarse_call(xp, kp)
    thr16 = thr.reshape(bsz, 128)[:, :16]
    cval, cidx, cq, cnt = _sc_compact(logits, q, thr16)

    ss = _CAP // 128
    samp, scal = _cand_call(cval.reshape(bsz, ss, 128),
                            cidx.reshape(bsz, ss, 128),
                            cq.reshape(bsz, ss, 128),
                            cnt, kp.reshape(bsz, 2), m.reshape(bsz, 128))
    logprobs = _final_call(logits, scal)
    return samp[:, 0], logprobs


_q_cache = {}


def kernel(logits, k, p):
    bsz, vocab = logits.shape
    if (bsz, vocab) not in _q_cache:
        try:
            with jax.ensure_compile_time_eval():
                _q_cache[(bsz, vocab)] = jax.random.exponential(
                    jax.random.key(42), (bsz, vocab), dtype=jnp.float32)
        except Exception:
            return _run(logits, k.astype(jnp.int32), p,
                        jax.random.exponential(jax.random.key(42),
                                               (bsz, vocab),
                                               dtype=jnp.float32))
    return _run(logits, k.astype(jnp.int32), p, _q_cache[(bsz, vocab)])


# speedup vs baseline: 3.3028x; 1.0827x over previous
import functools

import jax
import jax.numpy as jnp
import numpy as np
from jax import lax
from jax.experimental import pallas as pl
from jax.experimental.pallas import tpu as pltpu
from jax.experimental.pallas import tpu_sc as plsc

_NEG_INF = float("-inf")
_R = 8
_CAP = 4096
_CH = 2000
_COARSE_BITS = 14


def _monokey(x):
    b = lax.bitcast_convert_type(x, jnp.int32)
    return jnp.where(b >= 0, b, b ^ jnp.int32(0x7FFFFFFF))


def _ukey(key):
    return lax.bitcast_convert_type(key, jnp.uint32) ^ jnp.uint32(0x80000000)


def _coarse_kernel(x_ref, kp_ref, thr_ref, m_ref):
    x = x_ref[0]
    R, S, L = x.shape
    k16 = (_monokey(x) >> 16).astype(jnp.int16)

    M = jnp.max(x, axis=(1, 2), keepdims=True)
    kk = kp_ref[0][:, 0:1].astype(jnp.int32).reshape(R, 1, 1)
    one16, zero16 = jnp.int16(1), jnp.int16(0)

    def bit1(i, carry):
        lo, hi = carry
        mid = lo + (hi - lo + 1) // 2
        mid16 = mid.astype(jnp.int16)
        s16 = jnp.sum(jnp.where(k16 >= mid16, one16, zero16),
                      axis=1, keepdims=True)
        c = jnp.sum(s16.astype(jnp.int32), axis=2, keepdims=True)
        ok = c >= kk
        return jnp.where(ok, mid, lo), jnp.where(ok, hi, mid - 1)

    lo0 = jnp.full((R, 1, 1), -(1 << 15), jnp.int32)
    hi0 = jnp.full((R, 1, 1), (1 << 15) - 1, jnp.int32)
    lo, _ = lax.fori_loop(0, _COARSE_BITS, bit1, (lo0, hi0))
    ks = lo << 16
    bb = jnp.where(ks >= 0, ks, ks ^ jnp.int32(0x7FFFFFFF))
    thr_f = lax.bitcast_convert_type(bb, jnp.float32)
    thr_ref[0] = jnp.broadcast_to(thr_f.reshape(R, 1), (R, L))
    m_ref[0] = jnp.broadcast_to(M.reshape(R, 1), (R, L))


def _coarse_call(xp, kp, interpret=False):
    ng = xp.shape[0]
    R, sub = xp.shape[1], xp.shape[2]
    return pl.pallas_call(
        _coarse_kernel,
        grid=(ng,),
        in_specs=[
            pl.BlockSpec((1, R, sub, 128), lambda i: (i, 0, 0, 0)),
            pl.BlockSpec((1, R, 2), lambda i: (i, 0, 0)),
        ],
        out_specs=[
            pl.BlockSpec((1, R, 128), lambda i: (i, 0, 0)),
            pl.BlockSpec((1, R, 128), lambda i: (i, 0, 0)),
        ],
        out_shape=[
            jax.ShapeDtypeStruct((ng, R, 128), jnp.float32),
            jax.ShapeDtypeStruct((ng, R, 128), jnp.float32),
        ],
        interpret=interpret,
    )(xp, kp)


def _sc_compact(x, q, thr16):
    bsz, vocab = x.shape
    nchunk = vocab // _CH
    mesh = plsc.VectorSubcoreMesh(core_axis_name="c", subcore_axis_name="s")

    def mo(v):
        return pl.multiple_of(v, 8)

    @functools.partial(
        pl.kernel,
        mesh=mesh,
        compiler_params=pltpu.CompilerParams(needs_layout_passes=False),
        out_type=[
            jax.ShapeDtypeStruct((bsz * _CAP,), jnp.float32),
            jax.ShapeDtypeStruct((bsz * _CAP,), jnp.int32),
            jax.ShapeDtypeStruct((bsz * _CAP,), jnp.float32),
            jax.ShapeDtypeStruct((bsz * 16,), jnp.int32),
        ],
        scratch_types=[
            pltpu.VMEM((_CH,), jnp.float32),
            pltpu.VMEM((_CH,), jnp.float32),
            pltpu.VMEM((_CH,), jnp.float32),
            pltpu.VMEM((_CH,), jnp.float32),
            pltpu.VMEM((16,), jnp.float32),
            pltpu.VMEM((_CAP,), jnp.float32),
            pltpu.VMEM((_CAP,), jnp.int32),
            pltpu.VMEM((_CAP,), jnp.float32),
            pltpu.VMEM((16,), jnp.int32),
            pltpu.SemaphoreType.DMA,
            pltpu.SemaphoreType.DMA,
            pltpu.SemaphoreType.DMA,
            pltpu.SemaphoreType.DMA,
        ],
    )
    def body(x_hbm, q_hbm, thr_hbm, cval_hbm, cidx_hbm, cq_hbm, cnt_hbm,
             xa, xb, qa, qb, thrbuf, vbuf, ibuf, qcbuf, cntbuf,
             sxa, sxb, sqa, sqb):
        wid = lax.axis_index("s") * 2 + lax.axis_index("c")
        nhalf = nchunk // 2

        def fire(c, dx, dq, sx, sq, row):
            src = pl.ds(mo(row * vocab + c * _CH), _CH)
            cx = pltpu.make_async_copy(x_hbm.at[src], dx, sx)
            cq_ = pltpu.make_async_copy(q_hbm.at[src], dq, sq)
            cx.start()
            cq_.start()

        def wait(dx, dq, sx, sq):
            pltpu.make_async_copy(x_hbm.at[pl.ds(0, _CH)], dx, sx).wait()
            pltpu.make_async_copy(q_hbm.at[pl.ds(0, _CH)], dq, sq).wait()

        def process(xbuf, qbuf, base, thr, off):
            ofl = jnp.minimum(off, _CAP - _CH - 16)
            for i in range(_CH // 16):
                v = xbuf[pl.ds(i * 16, 16)]
                qv = qbuf[pl.ds(i * 16, 16)]
                m = v >= thr
                plsc.store_compressed(vbuf.at[pl.ds(ofl, 16)], v, mask=m)
                iv = lax.iota(jnp.int32, 16) + (base + i * 16)
                plsc.store_compressed(ibuf.at[pl.ds(ofl, 16)], iv, mask=m)
                plsc.store_compressed(qcbuf.at[pl.ds(ofl, 16)], qv, mask=m)
                ofl = ofl + plsc.all_reduce_population_count(m)[0]
            return ofl

        def row_body(rr, _):
            row = wid * 2 + rr
            pltpu.sync_copy(thr_hbm.at[pl.ds(mo(row * 16), 16)], thrbuf)
            thr = thrbuf[...]
            fire(0, xa, qa, sxa, sqa, row)

            def pair_body(c, off):
                fire(2 * c + 1, xb, qb, sxb, sqb, row)
                wait(xa, qa, sxa, sqa)
                off = process(xa, qa, 2 * c * _CH, thr, off)
                fire(jnp.minimum(2 * c + 2, nchunk - 1), xa, qa, sxa, sqa,
                     row)
                wait(xb, qb, sxb, sqb)
                off = process(xb, qb, (2 * c + 1) * _CH, thr, off)
                return off

            off = lax.fori_loop(0, nhalf, pair_body, jnp.int32(0))
            wait(xa, qa, sxa, sqa)
            pltpu.sync_copy(vbuf, cval_hbm.at[pl.ds(mo(row * _CAP), _CAP)])
            pltpu.sync_copy(ibuf, cidx_hbm.at[pl.ds(mo(row * _CAP), _CAP)])
            pltpu.sync_copy(qcbuf, cq_hbm.at[pl.ds(mo(row * _CAP), _CAP)])
            cntbuf[...] = jnp.zeros((16,), jnp.int32) + jnp.minimum(off, _CAP)
            pltpu.sync_copy(cntbuf, cnt_hbm.at[pl.ds(mo(row * 16), 16)])
            return jnp.int32(0)

        lax.fori_loop(0, 2, row_body, jnp.int32(0))

    cval, cidx, cq, cnt = body(x.reshape(-1), q.reshape(-1),
                               thr16.reshape(-1))
    return (cval.reshape(bsz, _CAP), cidx.reshape(bsz, _CAP),
            cq.reshape(bsz, _CAP), cnt.reshape(bsz, 16))


def _cand_kernel(cval_ref, cidx_ref, cq_ref, cnt_ref, kp_ref, m_ref,
                 samp_ref, scal_ref):
    val = cval_ref[...]
    ci = cidx_ref[...]
    cq = cq_ref[...]
    B, SS, L = val.shape

    slot = (lax.broadcasted_iota(jnp.int32, (B, SS, L), 1) * L
            + lax.broadcasted_iota(jnp.int32, (B, SS, L), 2))
    cnt = cnt_ref[...][:, 0:1].reshape(B, 1, 1)
    valid = slot < cnt

    keyc = _monokey(val)
    kbc = _ukey(keyc)

    def rsum(v):
        return jnp.sum(v, axis=(1, 2), keepdims=True)

    def rmax(v):
        return jnp.max(v, axis=(1, 2), keepdims=True)

    kk = kp_ref[...][:, 0:1].astype(jnp.int32).reshape(B, 1, 1)
    pp = kp_ref[...][:, 1:2].reshape(B, 1, 1)
    M = m_ref[...][:, 0:1].reshape(B, 1, 1)
    one, zero = jnp.int32(1), jnp.int32(0)

    def bit1(i, t):
        cand = t | (jnp.uint32(1) << (jnp.uint32(31) - i.astype(jnp.uint32)))
        c = rsum(jnp.where(valid & (kbc >= cand), one, zero))
        return jnp.where(c >= kk, cand, t)

    tk = lax.fori_loop(0, 32, bit1, jnp.zeros((B, 1, 1), jnp.uint32))

    e = jnp.where(valid, jnp.exp(val - M), 0.0)
    e_surv = jnp.where(kbc >= tk, e, 0.0)
    s1 = rsum(e_surv)
    target = pp * s1

    def bit2(i, t):
        bit = jnp.uint32(1) << (jnp.uint32(31) - i.astype(jnp.uint32))
        test = t | (bit - jnp.uint32(1))
        g = rsum(jnp.where(kbc > test, e_surv, 0.0))
        return jnp.where(g < target, t, t | bit)

    tp = lax.fori_loop(0, 32, bit2, jnp.zeros((B, 1, 1), jnp.uint32))

    tie = valid & (kbc == tp)
    e_star = rsum(jnp.where(kbc > tp, e_surv, 0.0))
    e_t = rmax(jnp.where(tie, e, 0.0))
    c_tie = rsum(jnp.where(tie, one, zero))

    jj = (lax.broadcasted_iota(jnp.int32, (1, 8, L), 1) * L
          + lax.broadcasted_iota(jnp.int32, (1, 8, L), 2)).astype(jnp.float32)
    need = jnp.sum(jnp.where(jj * e_t + e_star < target, one, zero),
                   axis=(1, 2), keepdims=True)
    d = (target - e_star) / jnp.maximum(e_t, jnp.float32(1e-37))
    d = jnp.minimum(d, jnp.float32(2e9))
    fl = jnp.floor(d)
    need_ar = fl.astype(jnp.int32) + jnp.where(d > fl, one, zero)
    need = jnp.where(need >= 8 * L, need_ar, need)
    need = jnp.minimum(need, c_tie)

    big = jnp.int32(1 << 30)

    def tie_bit(i, t):
        cand = t | (one << (jnp.int32(16) - i))
        c = rsum(jnp.where(tie & (ci >= cand), one, zero))
        return jnp.where(c >= need, cand, t)

    istar = lax.fori_loop(0, 17, tie_bit, jnp.zeros((B, 1, 1), jnp.int32))

    kmax = rmax(jnp.where(valid, keyc, jnp.int32(-(1 << 31))))
    ilast = rmax(jnp.where(valid & (keyc == kmax), ci, jnp.int32(-1)))

    kept = valid & ((kbc > tp) | (tie & (ci >= istar)) | (ci == ilast))
    s_kept = rsum(jnp.where(kept, e, 0.0))
    log_z = M + jnp.log(s_kept)

    score = jnp.where(kept, (e / s_kept) / cq, -1.0)
    smax = rmax(score)
    samp = jnp.min(jnp.where(score == smax, ci, big), axis=(1, 2),
                   keepdims=True)
    samp_ref[...] = jnp.broadcast_to(samp.reshape(B, 1), (B, 128))

    ks_tp = lax.bitcast_convert_type(tp ^ jnp.uint32(0x80000000), jnp.int32)
    scal = jnp.concatenate([
        lax.bitcast_convert_type(ks_tp, jnp.float32).reshape(B, 1),
        istar.astype(jnp.float32).reshape(B, 1),
        ilast.astype(jnp.float32).reshape(B, 1),
        log_z.reshape(B, 1),
    ], axis=1)
    scal_ref[...] = scal


def _cand_call(cval, cidx, cq, cnt, kp, m, interpret=False):
    B = cval.shape[0]
    return pl.pallas_call(
        _cand_kernel,
        out_shape=[
            jax.ShapeDtypeStruct((B, 128), jnp.int32),
            jax.ShapeDtypeStruct((B, 4), jnp.float32),
        ],
        interpret=interpret,
    )(cval, cidx, cq, cnt, kp, m)


def _final_kernel(x_ref, scal_ref, out_ref):
    x = x_ref[...]
    R, V = x.shape
    key = _monokey(x)
    idx = lax.broadcasted_iota(jnp.int32, (R, V), 1)
    sc = scal_ref[...]
    ks_tp = lax.bitcast_convert_type(sc[:, 0:1], jnp.int32)
    istar = sc[:, 1:2].astype(jnp.int32)
    ilast = sc[:, 2:3].astype(jnp.int32)
    log_z = sc[:, 3:4]
    kept = (key > ks_tp) | ((key == ks_tp) & (idx >= istar)) | (idx == ilast)
    out_ref[...] = jnp.where(kept, x - log_z, _NEG_INF)


def _final_call(x, scal, interpret=False):
    bsz, vocab = x.shape
    ng = bsz // _R
    return pl.pallas_call(
        _final_kernel,
        grid=(ng,),
        in_specs=[
            pl.BlockSpec((_R, vocab), lambda i: (i, 0)),
            pl.BlockSpec((_R, 4), lambda i: (i, 0)),
        ],
        out_specs=pl.BlockSpec((_R, vocab), lambda i: (i, 0)),
        out_shape=jax.ShapeDtypeStruct((bsz, vocab), jnp.float32),
        interpret=interpret,
    )(x, scal)


@jax.jit
def _run(logits, k, p, q):
    bsz, vocab = logits.shape
    pv = ((vocab + 1023) // 1024) * 1024
    sub = pv // 128
    ng = bsz // _R
    xp = jnp.pad(logits, ((0, 0), (0, pv - vocab)),
                 constant_values=_NEG_INF).reshape(ng, _R, sub, 128)
    kp = jnp.stack([k.astype(jnp.float32), p], axis=-1).reshape(ng, _R, 2)

    thr, m = _coarse_call(xp, kp)
    thr16 = thr.reshape(bsz, 128)[:, :16]
    cval, cidx, cq, cnt = _sc_compact(logits, q, thr16)

    ss = _CAP // 128
    samp, scal = _cand_call(cval.reshape(bsz, ss, 128),
                            cidx.reshape(bsz, ss, 128),
                            cq.reshape(bsz, ss, 128),
                            cnt, kp.reshape(bsz, 2), m.reshape(bsz, 128))
    logprobs = _final_call(logits, scal)
    return samp[:, 0], logprobs


_q_cache = {}


def kernel(logits, k, p):
    bsz, vocab = logits.shape
    if (bsz, vocab) not in _q_cache:
        try:
            with jax.ensure_compile_time_eval():
                _q_cache[(bsz, vocab)] = jax.random.exponential(
                    jax.random.key(42), (bsz, vocab), dtype=jnp.float32)
        except Exception:
            return _run(logits, k.astype(jnp.int32), p,
                        jax.random.exponential(jax.random.key(42),
                                               (bsz, vocab),
                                               dtype=jnp.float32))
    return _run(logits, k.astype(jnp.int32), p, _q_cache[(bsz, vocab)])
